# Initial kernel scaffold; baseline (speedup 1.0000x reference)
import jax
import jax.numpy as jnp
from jax.experimental import pallas as pl


def kernel(x, edge_index, W_gc, b_gc, bn_gamma, bn_beta, fc_W, fc_b):
    raise NotImplementedError("write your pallas kernel here")


# speedup vs baseline: 12.4070x; 12.4070x over previous
import functools

import jax
import jax.numpy as jnp
from jax import lax
from jax.experimental import pallas as pl
from jax.experimental.pallas import tpu as pltpu
from jax.experimental.pallas import tpu_sc as plsc

N = 10000
F = 128
NCLASS = 2
NC, NS, L = 2, 16, 16
NW = NC * NS
K = 64
SB_CH = 32
SB = 5
CH = SB * SB_CH
E_PAD = NW * CH * K
ROWS_PER_TILE = 640
N_PAD = NS * ROWS_PER_TILE
DEG_W = 16

_mesh = plsc.VectorSubcoreMesh(
    core_axis_name="c", subcore_axis_name="s", num_cores=NC, num_subcores=NS)


@functools.partial(
    pl.kernel,
    out_type=jax.ShapeDtypeStruct((NC, N_PAD, DEG_W), jnp.float32),
    mesh=_mesh,
    scratch_types=[
        pltpu.VMEM((SB_CH, K), jnp.int32),
        pltpu.VMEM((K, DEG_W), jnp.float32),
        pltpu.VMEM((K, DEG_W), jnp.float32),
        pltpu.VMEM_SHARED((N_PAD, DEG_W), jnp.float32),
        pltpu.SemaphoreType.DMA,
    ],
)
def _deg_kernel(col_hbm, out_hbm, cidx, ones_v, zero_v, deg_sh, sem_s):
    c = lax.axis_index("c")
    s = lax.axis_index("s")
    wid = s * NC + c

    def fill(i, carry):
        ones_v[i] = jnp.ones((DEG_W,), jnp.float32)
        zero_v[i] = jnp.zeros((DEG_W,), jnp.float32)
        return carry
    lax.fori_loop(0, K, fill, 0)

    base = s * ROWS_PER_TILE
    for p in range(ROWS_PER_TILE // K):
        pltpu.sync_copy(zero_v, deg_sh.at[pl.ds(base + p * K, K)])
    plsc.subcore_barrier()

    def sblock(p, carry):
        pltpu.sync_copy(col_hbm.at[wid, pl.ds(p * SB_CH, SB_CH)], cidx)

        def scat(q, c2):
            pltpu.async_copy(ones_v, deg_sh.at[cidx.at[q]], sem_s, add=True)
            return c2
        lax.fori_loop(0, SB_CH, scat, 0)

        def drain(q, c2):
            pltpu.make_async_copy(ones_v, deg_sh.at[cidx.at[q]], sem_s).wait()
            return c2
        lax.fori_loop(0, SB_CH, drain, 0)
        return carry
    lax.fori_loop(0, SB, sblock, 0)

    plsc.subcore_barrier()
    pltpu.sync_copy(deg_sh.at[pl.ds(base, ROWS_PER_TILE)],
                    out_hbm.at[c, pl.ds(base, ROWS_PER_TILE)])


@functools.partial(
    pl.kernel,
    out_type=jax.ShapeDtypeStruct((NC, N_PAD, F), jnp.float32),
    mesh=_mesh,
    scratch_types=[
        pltpu.VMEM((SB_CH, K), jnp.int32),
        pltpu.VMEM((SB_CH, K), jnp.int32),
        pltpu.VMEM((K, F), jnp.float32),
        pltpu.VMEM((K, F), jnp.float32),
        pltpu.VMEM_SHARED((N_PAD, F), jnp.float32),
        pltpu.SemaphoreType.DMA,
        pltpu.SemaphoreType.DMA,
        pltpu.SemaphoreType.DMA,
        pltpu.SemaphoreType.DMA,
    ],
)
def _edge_kernel(row_hbm, col_hbm, g_hbm, out_hbm, ridx, cidx, buf0, buf1,
                 acc_sh, gs0, gs1, ss0, ss1):
    c = lax.axis_index("c")
    s = lax.axis_index("s")
    wid = s * NC + c

    def zrow(i, carry):
        for kk in range(F // L):
            buf0[i, pl.ds(kk * L, L)] = jnp.zeros((L,), jnp.float32)
        return carry
    lax.fori_loop(0, K, zrow, 0)
    base = s * ROWS_PER_TILE
    for p in range(ROWS_PER_TILE // K):
        pltpu.sync_copy(buf0, acc_sh.at[pl.ds(base + p * K, K)])
    plsc.subcore_barrier()

    bufs = (buf0, buf1)
    gsem = (gs0, gs1)
    ssem = (ss0, ss1)

    def sblock(p, carry):
        pltpu.sync_copy(row_hbm.at[wid, pl.ds(p * SB_CH, SB_CH)], ridx)
        pltpu.sync_copy(col_hbm.at[wid, pl.ds(p * SB_CH, SB_CH)], cidx)
        pltpu.async_copy(g_hbm.at[ridx.at[0]], buf0, gs0)
        pltpu.async_copy(g_hbm.at[ridx.at[1]], buf1, gs1)

        def body(i, c2):
            q0 = i * 2
            for b in range(2):
                q = q0 + b
                pltpu.make_async_copy(
                    g_hbm.at[ridx.at[q]], bufs[b], gsem[b]).wait()
                pltpu.async_copy(
                    bufs[b], acc_sh.at[cidx.at[q]], ssem[b], add=True)

                @pl.when(q + 2 < SB_CH)
                def _():
                    pltpu.make_async_copy(
                        bufs[b], acc_sh.at[cidx.at[q]], ssem[b]).wait()
                    pltpu.async_copy(g_hbm.at[ridx.at[q + 2]], bufs[b], gsem[b])
            return c2
        lax.fori_loop(0, SB_CH // 2, body, 0)
        pltpu.make_async_copy(buf0, acc_sh.at[cidx.at[SB_CH - 2]], ss0).wait()
        pltpu.make_async_copy(buf1, acc_sh.at[cidx.at[SB_CH - 1]], ss1).wait()
        return carry
    lax.fori_loop(0, SB, sblock, 0)

    plsc.subcore_barrier()
    pltpu.sync_copy(acc_sh.at[pl.ds(base, ROWS_PER_TILE)],
                    out_hbm.at[c, pl.ds(base, ROWS_PER_TILE)])


def _prep_body(x_ref, w_ref, degp_ref, g_ref):
    deg = degp_ref[0, :, 0:1] + degp_ref[1, :, 0:1] + 1.0
    dis = lax.rsqrt(deg)
    h = jnp.dot(x_ref[...], w_ref[...], preferred_element_type=jnp.float32)
    g_ref[pl.ds(0, N), :] = h * dis[0:N]
    g_ref[pl.ds(N, N_PAD - N), :] = jnp.zeros((N_PAD - N, F), jnp.float32)


def _final_body(accp_ref, g_ref, degp_ref, b_ref, gam_ref, bet_ref,
                fcw_ref, fcb_ref, logits_ref, embed_ref):
    deg = degp_ref[0, :, 0:1] + degp_ref[1, :, 0:1] + 1.0
    dis = lax.rsqrt(deg[0:N])
    ssum = accp_ref[0, 0:N, :] + accp_ref[1, 0:N, :] + g_ref[0:N, :]
    out = ssum * dis + b_ref[...]
    mean = jnp.mean(out, axis=0, keepdims=True)
    cent = out - mean
    var = jnp.mean(cent * cent, axis=0, keepdims=True)
    embed = cent * lax.rsqrt(var + 1e-5) * gam_ref[...] + bet_ref[...]
    embed_ref[...] = embed
    logits_ref[...] = jnp.dot(embed[:, 0:F // 2], fcw_ref[...],
                              preferred_element_type=jnp.float32) + fcb_ref[...]


def kernel(x, edge_index, W_gc, b_gc, bn_gamma, bn_beta, fc_W, fc_b):
    E = edge_index.shape[1]
    row = edge_index[0].astype(jnp.int32)
    col = edge_index[1].astype(jnp.int32)
    padv = jnp.full((E_PAD - E,), N, jnp.int32)
    row_t = jnp.concatenate([row, padv]).reshape(NW, CH, K)
    col_t = jnp.concatenate([col, padv]).reshape(NW, CH, K)

    degp = _deg_kernel(col_t)

    g_pad = pl.pallas_call(
        _prep_body,
        out_shape=jax.ShapeDtypeStruct((N_PAD, F), jnp.float32),
    )(x, W_gc, degp)

    accp = _edge_kernel(row_t, col_t, g_pad)

    logits, embed = pl.pallas_call(
        _final_body,
        out_shape=[
            jax.ShapeDtypeStruct((N, NCLASS), jnp.float32),
            jax.ShapeDtypeStruct((N, F), jnp.float32),
        ],
    )(accp, g_pad, degp, b_gc.reshape(1, F), bn_gamma.reshape(1, F),
      bn_beta.reshape(1, F), fc_W, fc_b.reshape(1, NCLASS))
    return (logits, embed)
